# baseline (speedup 1.0000x reference)
import functools

import jax
import jax.numpy as jnp
from jax import lax
from jax.experimental import pallas as pl
from jax.experimental.pallas import tpu as pltpu
from jax.experimental.pallas import tpu_sc as plsc

NE = 10000
NR = 16
ED = 128
RD = 64
B = 320000
REG_LAMBDA = 0.01

NW = 32
TPW = B // NW
CH = 80
NCH = TPW // CH

_C0 = 0.6934594480030593
_C1 = 0.12327823655443833
_C2 = -0.003782233978508922

EBLK = 2000


def _proj_body(ent_ref, m2_ref, re2_ref, out_ref):
    e = ent_ref[...]
    m2 = m2_ref[0]
    p2 = jnp.dot(e, m2, preferred_element_type=jnp.float32)
    out_ref[0] = p2 + re2_ref[pl.program_id(1)][None, :]


def _norm_body(ent3_ref, rel3_ref, oe_ref, orl_ref):
    x = ent3_ref[...]
    oe_ref[...] = jnp.sqrt(jnp.sum(x * x, axis=-1))
    rl = rel3_ref[...]
    orl_ref[...] = jnp.sqrt(jnp.sum(rl * rl, axis=-1))


def _poly_softplus_acc(a, tt, tp, pacc, xacc):
    d1 = a - tt
    d2 = a - tp
    x = d2 * d2 - d1 * d1
    y = x * x
    p = _C2
    p = p * y + _C1
    p = p * y + _C0
    return pacc + p, xacc + x


def _sc_body(proj_hbm, en_hbm, rn_hbm, h_hbm, r_hbm, t_hbm,
             tp_hbm, out_hbm,
             hj, rj, tj, tpj,
             ihA, itA, itpA, rowhA, rowtA, rowtpA,
             ihB, itB, itpB, rowhB, rowtB, rowtpB,
             en_v, rn_v, outb, semA, semB):
    wid = lax.axis_index("s") * 2 + lax.axis_index("c")
    base = wid * TPW

    pltpu.sync_copy(h_hbm.at[pl.ds(base, TPW)], hj)
    pltpu.sync_copy(r_hbm.at[pl.ds(base, TPW)], rj)
    pltpu.sync_copy(t_hbm.at[pl.ds(base, TPW)], tj)
    pltpu.sync_copy(tp_hbm.at[pl.ds(base, TPW)], tpj)
    pltpu.sync_copy(en_hbm, en_v)
    pltpu.sync_copy(rn_hbm, rn_v)

    zf = jnp.zeros((16,), jnp.float32)

    bufs = {
        0: (ihA, itA, itpA, rowhA, rowtA, rowtpA, semA),
        1: (ihB, itB, itpB, rowhB, rowtB, rowtpB, semB),
    }

    def fire(c, b, nacc):
        ih, it, itp, rowh, rowt, rowtp, sem = bufs[b]
        coff = c * CH

        def g_body(g, nacc_in):
            s = coff + g * 16
            so = g * 16
            hv = hj[pl.ds(s, 16)]
            rv = rj[pl.ds(s, 16)]
            tv = tj[pl.ds(s, 16)]
            tpv = tpj[pl.ds(s, 16)]
            m2 = rv * (2 * NE)
            ih[pl.ds(so, 16)] = m2 + (hv + hv)
            it[pl.ds(so, 16)] = m2 + (tv + tv) + 1
            itp[pl.ds(so, 16)] = m2 + (tpv + tpv) + 1
            nh = plsc.load_gather(en_v, [hv])
            nt = plsc.load_gather(en_v, [tv])
            ntp = plsc.load_gather(en_v, [tpv])
            nr = plsc.load_gather(rn_v, [rv])
            return nacc_in + ((nh + nt) + (ntp + nr))

        nacc = lax.fori_loop(0, CH // 16, g_body, nacc)
        pltpu.async_copy(proj_hbm.at[ih], rowh, sem)
        pltpu.async_copy(proj_hbm.at[it], rowt, sem)
        pltpu.async_copy(proj_hbm.at[itp], rowtp, sem)
        return nacc

    def wait_and_compute(b, pacc, xacc):
        ih, it, itp, rowh, rowt, rowtp, sem = bufs[b]
        pltpu.make_async_copy(proj_hbm.at[ih], rowh, sem).wait()
        pltpu.make_async_copy(proj_hbm.at[it], rowt, sem).wait()
        pltpu.make_async_copy(proj_hbm.at[itp], rowtp, sem).wait()

        @plsc.parallel_loop(0, CH, unroll=4, carry=(pacc, xacc))
        def j_loop(j, pc_xc):
            pc, xc = pc_xc
            for kk in range(RD // 16):
                a = rowh[j, pl.ds(kk * 16, 16)]
                tt = rowt[j, pl.ds(kk * 16, 16)]
                tp = rowtp[j, pl.ds(kk * 16, 16)]
                pc, xc = _poly_softplus_acc(a, tt, tp, pc, xc)
            return (pc, xc)

        return j_loop

    nacc = fire(0, 0, zf)

    def pair_body(p, carry):
        pacc, xacc, nacc = carry
        c0 = 2 * p
        nacc = fire(c0 + 1, 1, nacc)
        pacc, xacc = wait_and_compute(0, pacc, xacc)
        nacc = fire(c0 + 2, 0, nacc)
        pacc, xacc = wait_and_compute(1, pacc, xacc)
        return (pacc, xacc, nacc)

    pacc, xacc, nacc = lax.fori_loop(0, (NCH - 1) // 2, pair_body,
                                     (zf, zf, nacc))
    pacc, xacc = wait_and_compute(0, pacc, xacc)

    outb[pl.ds(0, 16)] = pacc
    outb[pl.ds(16, 16)] = xacc
    outb[pl.ds(32, 16)] = nacc
    outb[pl.ds(48, 16)] = zf
    pltpu.sync_copy(outb, out_hbm.at[wid])


def kernel(h, r, t, t_prime, entity_emb, relation_emb, transformation_M):
    h = h.astype(jnp.int32)
    r = r.astype(jnp.int32)
    t = t.astype(jnp.int32)
    t_prime = t_prime.astype(jnp.int32)

    m2 = jnp.concatenate([transformation_M, transformation_M], axis=-1)
    re2 = jnp.concatenate(
        [relation_emb, jnp.zeros((NR, RD), jnp.float32)], axis=-1)
    proj3 = pl.pallas_call(
        _proj_body,
        grid=(NE // EBLK, NR),
        in_specs=[
            pl.BlockSpec((EBLK, ED), lambda i, j: (i, 0)),
            pl.BlockSpec((1, ED, 2 * RD), lambda i, j: (j, 0, 0)),
            pl.BlockSpec((NR, 2 * RD), lambda i, j: (0, 0)),
        ],
        out_specs=pl.BlockSpec((1, EBLK, 2 * RD), lambda i, j: (j, i, 0)),
        out_shape=jax.ShapeDtypeStruct((NR, NE, 2 * RD), jnp.float32),
    )(entity_emb, m2, re2)
    proj = proj3.reshape(2 * NR * NE, RD)

    en_tab, rn_tab = pl.pallas_call(
        _norm_body,
        out_shape=[
            jax.ShapeDtypeStruct((80, 125), jnp.float32),
            jax.ShapeDtypeStruct((1, 16), jnp.float32),
        ],
    )(entity_emb.reshape(80, 125, ED), relation_emb.reshape(1, NR, RD))

    mesh = plsc.VectorSubcoreMesh(core_axis_name="c", subcore_axis_name="s")
    parts = pl.kernel(
        _sc_body,
        mesh=mesh,
        compiler_params=pltpu.CompilerParams(needs_layout_passes=False,
                                             use_tc_tiling_on_sc=False),
        out_type=jax.ShapeDtypeStruct((NW, 64), jnp.float32),
        scratch_types=[
            pltpu.VMEM((TPW,), jnp.int32),
            pltpu.VMEM((TPW,), jnp.int32),
            pltpu.VMEM((TPW,), jnp.int32),
            pltpu.VMEM((TPW,), jnp.int32),
            pltpu.VMEM((CH,), jnp.int32),
            pltpu.VMEM((CH,), jnp.int32),
            pltpu.VMEM((CH,), jnp.int32),
            pltpu.VMEM((CH, RD), jnp.float32),
            pltpu.VMEM((CH, RD), jnp.float32),
            pltpu.VMEM((CH, RD), jnp.float32),
            pltpu.VMEM((CH,), jnp.int32),
            pltpu.VMEM((CH,), jnp.int32),
            pltpu.VMEM((CH,), jnp.int32),
            pltpu.VMEM((CH, RD), jnp.float32),
            pltpu.VMEM((CH, RD), jnp.float32),
            pltpu.VMEM((CH, RD), jnp.float32),
            pltpu.VMEM((NE,), jnp.float32),
            pltpu.VMEM((NR,), jnp.float32),
            pltpu.VMEM((64,), jnp.float32),
            pltpu.SemaphoreType.DMA,
            pltpu.SemaphoreType.DMA,
        ],
    )(proj, en_tab.reshape(NE), rn_tab.reshape(NR), h, r,
      t, t_prime)

    sum_poly = jnp.sum(parts[:, 0:16])
    sum_x = jnp.sum(parts[:, 16:32])
    sum_norm = jnp.sum(parts[:, 32:48])
    loss = (sum_poly - 0.5 * sum_x) / jnp.float32(B * RD)
    reg = sum_norm / jnp.float32(B)
    return (loss + REG_LAMBDA * reg).astype(jnp.float32)


# speedup vs baseline: 1.0303x; 1.0303x over previous
import functools

import jax
import jax.numpy as jnp
from jax import lax
from jax.experimental import pallas as pl
from jax.experimental.pallas import tpu as pltpu
from jax.experimental.pallas import tpu_sc as plsc

NE = 10000
NR = 16
ED = 128
RD = 64
B = 320000
REG_LAMBDA = 0.01

NW = 32
TPW = B // NW
CH = 80
NCH = TPW // CH

_C0 = 0.6934594480030593
_C1 = 0.12327823655443833
_C2 = -0.003782233978508922

EBLK = 2000


def _proj_body(ent_ref, m2_ref, re2_ref, out_ref):
    e = ent_ref[...]
    m2 = m2_ref[0]
    p2 = jnp.dot(e, m2, preferred_element_type=jnp.float32)
    out_ref[0] = p2 + re2_ref[pl.program_id(1)][None, :]


def _norm_body(ent3_ref, rel3_ref, oe_ref, orl_ref):
    x = ent3_ref[...]
    oe_ref[...] = jnp.sqrt(jnp.sum(x * x, axis=-1))
    rl = rel3_ref[...]
    orl_ref[...] = jnp.sqrt(jnp.sum(rl * rl, axis=-1))


def _poly_softplus_acc(a, tt, tp, pacc, xacc):
    d1 = a - tt
    d2 = a - tp
    x = d2 * d2 - d1 * d1
    y = x * x
    p = _C2
    p = p * y + _C1
    p = p * y + _C0
    return pacc + p, xacc + x


def _sc_body(proj_hbm, en_hbm, rn_hbm, h_hbm, r_hbm, t_hbm,
             tp_hbm, out_hbm,
             hj, rj, tj, tpj,
             ihA, itA, itpA, rowhA, rowtA, rowtpA,
             ihB, itB, itpB, rowhB, rowtB, rowtpB,
             en_v, rn_v, outb, semA, semB):
    wid = lax.axis_index("s") * 2 + lax.axis_index("c")
    base = wid * TPW

    st1 = pltpu.async_copy(h_hbm.at[pl.ds(base, TPW)], hj, semA)
    st2 = pltpu.async_copy(r_hbm.at[pl.ds(base, TPW)], rj, semA)
    st3 = pltpu.async_copy(t_hbm.at[pl.ds(base, TPW)], tj, semA)
    st4 = pltpu.async_copy(tp_hbm.at[pl.ds(base, TPW)], tpj, semA)
    st5 = pltpu.async_copy(en_hbm, en_v, semA)
    st6 = pltpu.async_copy(rn_hbm, rn_v, semA)
    st1.wait()
    st2.wait()
    st3.wait()
    st4.wait()
    st5.wait()
    st6.wait()

    zf = jnp.zeros((16,), jnp.float32)

    bufs = {
        0: (ihA, itA, itpA, rowhA, rowtA, rowtpA, semA),
        1: (ihB, itB, itpB, rowhB, rowtB, rowtpB, semB),
    }

    def fire(c, b, nacc):
        ih, it, itp, rowh, rowt, rowtp, sem = bufs[b]
        coff = c * CH

        def g_body(g, nacc_in):
            s = coff + g * 16
            so = g * 16
            hv = hj[pl.ds(s, 16)]
            rv = rj[pl.ds(s, 16)]
            tv = tj[pl.ds(s, 16)]
            tpv = tpj[pl.ds(s, 16)]
            m2 = rv * (2 * NE)
            ih[pl.ds(so, 16)] = m2 + (hv + hv)
            it[pl.ds(so, 16)] = m2 + (tv + tv) + 1
            itp[pl.ds(so, 16)] = m2 + (tpv + tpv) + 1
            nh = plsc.load_gather(en_v, [hv])
            nt = plsc.load_gather(en_v, [tv])
            ntp = plsc.load_gather(en_v, [tpv])
            nr = plsc.load_gather(rn_v, [rv])
            return nacc_in + ((nh + nt) + (ntp + nr))

        nacc = lax.fori_loop(0, CH // 16, g_body, nacc)
        pltpu.async_copy(proj_hbm.at[ih], rowh, sem)
        pltpu.async_copy(proj_hbm.at[it], rowt, sem)
        pltpu.async_copy(proj_hbm.at[itp], rowtp, sem)
        return nacc

    def wait_and_compute(b, pacc, xacc):
        ih, it, itp, rowh, rowt, rowtp, sem = bufs[b]
        pltpu.make_async_copy(proj_hbm.at[ih], rowh, sem).wait()
        pltpu.make_async_copy(proj_hbm.at[it], rowt, sem).wait()
        pltpu.make_async_copy(proj_hbm.at[itp], rowtp, sem).wait()

        def j_body(j, pc_xc):
            pc, xc = pc_xc
            for kk in range(RD // 16):
                a = rowh[j, pl.ds(kk * 16, 16)]
                tt = rowt[j, pl.ds(kk * 16, 16)]
                tp = rowtp[j, pl.ds(kk * 16, 16)]
                pc, xc = _poly_softplus_acc(a, tt, tp, pc, xc)
            return (pc, xc)

        return lax.fori_loop(0, CH, j_body, (pacc, xacc))

    nacc = fire(0, 0, zf)

    def pair_body(p, carry):
        pacc, xacc, nacc = carry
        c0 = 2 * p
        nacc = fire(c0 + 1, 1, nacc)
        pacc, xacc = wait_and_compute(0, pacc, xacc)
        nacc = fire(c0 + 2, 0, nacc)
        pacc, xacc = wait_and_compute(1, pacc, xacc)
        return (pacc, xacc, nacc)

    pacc, xacc, nacc = lax.fori_loop(0, (NCH - 1) // 2, pair_body,
                                     (zf, zf, nacc))
    pacc, xacc = wait_and_compute(0, pacc, xacc)

    outb[pl.ds(0, 16)] = pacc
    outb[pl.ds(16, 16)] = xacc
    outb[pl.ds(32, 16)] = nacc
    outb[pl.ds(48, 16)] = zf
    pltpu.sync_copy(outb, out_hbm.at[wid])


def kernel(h, r, t, t_prime, entity_emb, relation_emb, transformation_M):
    h = h.astype(jnp.int32)
    r = r.astype(jnp.int32)
    t = t.astype(jnp.int32)
    t_prime = t_prime.astype(jnp.int32)

    m2 = jnp.concatenate([transformation_M, transformation_M], axis=-1)
    re2 = jnp.concatenate(
        [relation_emb, jnp.zeros((NR, RD), jnp.float32)], axis=-1)
    proj3 = pl.pallas_call(
        _proj_body,
        grid=(NE // EBLK, NR),
        in_specs=[
            pl.BlockSpec((EBLK, ED), lambda i, j: (i, 0)),
            pl.BlockSpec((1, ED, 2 * RD), lambda i, j: (j, 0, 0)),
            pl.BlockSpec((NR, 2 * RD), lambda i, j: (0, 0)),
        ],
        out_specs=pl.BlockSpec((1, EBLK, 2 * RD), lambda i, j: (j, i, 0)),
        out_shape=jax.ShapeDtypeStruct((NR, NE, 2 * RD), jnp.float32),
    )(entity_emb, m2, re2)
    proj = proj3.reshape(2 * NR * NE, RD)

    en_tab, rn_tab = pl.pallas_call(
        _norm_body,
        out_shape=[
            jax.ShapeDtypeStruct((80, 125), jnp.float32),
            jax.ShapeDtypeStruct((1, 16), jnp.float32),
        ],
    )(entity_emb.reshape(80, 125, ED), relation_emb.reshape(1, NR, RD))

    mesh = plsc.VectorSubcoreMesh(core_axis_name="c", subcore_axis_name="s")
    parts = pl.kernel(
        _sc_body,
        mesh=mesh,
        compiler_params=pltpu.CompilerParams(needs_layout_passes=False,
                                             use_tc_tiling_on_sc=False),
        out_type=jax.ShapeDtypeStruct((NW, 64), jnp.float32),
        scratch_types=[
            pltpu.VMEM((TPW,), jnp.int32),
            pltpu.VMEM((TPW,), jnp.int32),
            pltpu.VMEM((TPW,), jnp.int32),
            pltpu.VMEM((TPW,), jnp.int32),
            pltpu.VMEM((CH,), jnp.int32),
            pltpu.VMEM((CH,), jnp.int32),
            pltpu.VMEM((CH,), jnp.int32),
            pltpu.VMEM((CH, RD), jnp.float32),
            pltpu.VMEM((CH, RD), jnp.float32),
            pltpu.VMEM((CH, RD), jnp.float32),
            pltpu.VMEM((CH,), jnp.int32),
            pltpu.VMEM((CH,), jnp.int32),
            pltpu.VMEM((CH,), jnp.int32),
            pltpu.VMEM((CH, RD), jnp.float32),
            pltpu.VMEM((CH, RD), jnp.float32),
            pltpu.VMEM((CH, RD), jnp.float32),
            pltpu.VMEM((NE,), jnp.float32),
            pltpu.VMEM((NR,), jnp.float32),
            pltpu.VMEM((64,), jnp.float32),
            pltpu.SemaphoreType.DMA,
            pltpu.SemaphoreType.DMA,
        ],
    )(proj, en_tab.reshape(NE), rn_tab.reshape(NR), h, r,
      t, t_prime)

    sum_poly = jnp.sum(parts[:, 0:16])
    sum_x = jnp.sum(parts[:, 16:32])
    sum_norm = jnp.sum(parts[:, 32:48])
    loss = (sum_poly - 0.5 * sum_x) / jnp.float32(B * RD)
    reg = sum_norm / jnp.float32(B)
    return (loss + REG_LAMBDA * reg).astype(jnp.float32)


# speedup vs baseline: 1.0504x; 1.0195x over previous
import functools

import jax
import jax.numpy as jnp
from jax import lax
from jax.experimental import pallas as pl
from jax.experimental.pallas import tpu as pltpu
from jax.experimental.pallas import tpu_sc as plsc

NE = 10000
NR = 16
ED = 128
RD = 64
B = 320000
REG_LAMBDA = 0.01

NW = 32
TPW = B // NW
CH = 80
NCH = TPW // CH

_C0 = 0.6934594480030593
_C1 = 0.12327823655443833
_C2 = -0.003782233978508922

EBLK = 2000


def _proj_body(ent_ref, m2_ref, re2_ref, out_ref, nrm_ref):
    e = ent_ref[...]
    m2 = m2_ref[0]
    p2 = jnp.dot(e, m2, preferred_element_type=jnp.float32)
    out_ref[0] = p2 + re2_ref[pl.program_id(1)][None, :]

    @pl.when(pl.program_id(1) == 0)
    def _():
        n = jnp.sqrt(jnp.sum(e * e, axis=1))
        rl = re2_ref[...][:, :RD]
        rn = jnp.sqrt(jnp.sum(rl * rl, axis=1))
        tail = jnp.where(pl.program_id(0) == 0, rn, 0.0)
        row = jnp.concatenate([n, tail, jnp.zeros((32,), jnp.float32)])
        nrm_ref[...] = row.reshape(1, 1, 2048)


def _poly_softplus_acc(a, tt, tp, pacc, xacc):
    d1 = a - tt
    d2 = a - tp
    x = d2 * d2 - d1 * d1
    y = x * x
    p = _C2
    p = p * y + _C1
    p = p * y + _C0
    return pacc + p, xacc + x


def _sc_body(proj_hbm, en_hbm, h_hbm, r_hbm, t_hbm,
             tp_hbm, out_hbm,
             hj, rj, tj, tpj,
             ihA, itA, itpA, rowhA, rowtA, rowtpA,
             ihB, itB, itpB, rowhB, rowtB, rowtpB,
             en_v, outb, semA, semB):
    wid = lax.axis_index("s") * 2 + lax.axis_index("c")
    base = wid * TPW

    st1 = pltpu.async_copy(h_hbm.at[pl.ds(base, TPW)], hj, semA)
    st2 = pltpu.async_copy(r_hbm.at[pl.ds(base, TPW)], rj, semA)
    st3 = pltpu.async_copy(t_hbm.at[pl.ds(base, TPW)], tj, semA)
    st4 = pltpu.async_copy(tp_hbm.at[pl.ds(base, TPW)], tpj, semA)
    st5 = pltpu.async_copy(en_hbm, en_v, semA)
    st1.wait()
    st2.wait()
    st3.wait()
    st4.wait()
    st5.wait()

    zf = jnp.zeros((16,), jnp.float32)

    bufs = {
        0: (ihA, itA, itpA, rowhA, rowtA, rowtpA, semA),
        1: (ihB, itB, itpB, rowhB, rowtB, rowtpB, semB),
    }

    def fire(c, b, nacc):
        ih, it, itp, rowh, rowt, rowtp, sem = bufs[b]
        coff = c * CH

        def g_body(g, nacc_in):
            s = coff + g * 16
            so = g * 16
            hv = hj[pl.ds(s, 16)]
            rv = rj[pl.ds(s, 16)]
            tv = tj[pl.ds(s, 16)]
            tpv = tpj[pl.ds(s, 16)]
            m2 = rv * (2 * NE)
            ih[pl.ds(so, 16)] = m2 + (hv + hv)
            it[pl.ds(so, 16)] = m2 + (tv + tv) + 1
            itp[pl.ds(so, 16)] = m2 + (tpv + tpv) + 1
            def nidx(ev):
                q = jnp.right_shift(ev * 67109, 27)
                return jnp.left_shift(q, 11) + (ev - q * 2000)

            nh = plsc.load_gather(en_v, [nidx(hv)])
            nt = plsc.load_gather(en_v, [nidx(tv)])
            ntp = plsc.load_gather(en_v, [nidx(tpv)])
            nr = plsc.load_gather(en_v, [rv + 2000])
            return nacc_in + ((nh + nt) + (ntp + nr))

        nacc = lax.fori_loop(0, CH // 16, g_body, nacc)
        pltpu.async_copy(proj_hbm.at[ih], rowh, sem)
        pltpu.async_copy(proj_hbm.at[it], rowt, sem)
        pltpu.async_copy(proj_hbm.at[itp], rowtp, sem)
        return nacc

    def wait_and_compute(b, pacc, xacc):
        ih, it, itp, rowh, rowt, rowtp, sem = bufs[b]
        pltpu.make_async_copy(proj_hbm.at[ih], rowh, sem).wait()
        pltpu.make_async_copy(proj_hbm.at[it], rowt, sem).wait()
        pltpu.make_async_copy(proj_hbm.at[itp], rowtp, sem).wait()

        def j_body(j, pc_xc):
            pc, xc = pc_xc
            for kk in range(RD // 16):
                a = rowh[j, pl.ds(kk * 16, 16)]
                tt = rowt[j, pl.ds(kk * 16, 16)]
                tp = rowtp[j, pl.ds(kk * 16, 16)]
                pc, xc = _poly_softplus_acc(a, tt, tp, pc, xc)
            return (pc, xc)

        return lax.fori_loop(0, CH, j_body, (pacc, xacc))

    nacc = fire(0, 0, zf)

    def pair_body(p, carry):
        pacc, xacc, nacc = carry
        c0 = 2 * p
        nacc = fire(c0 + 1, 1, nacc)
        pacc, xacc = wait_and_compute(0, pacc, xacc)
        nacc = fire(c0 + 2, 0, nacc)
        pacc, xacc = wait_and_compute(1, pacc, xacc)
        return (pacc, xacc, nacc)

    pacc, xacc, nacc = lax.fori_loop(0, (NCH - 1) // 2, pair_body,
                                     (zf, zf, nacc))
    pacc, xacc = wait_and_compute(0, pacc, xacc)

    outb[pl.ds(0, 16)] = pacc
    outb[pl.ds(16, 16)] = xacc
    outb[pl.ds(32, 16)] = nacc
    outb[pl.ds(48, 16)] = zf
    pltpu.sync_copy(outb, out_hbm.at[wid])


def kernel(h, r, t, t_prime, entity_emb, relation_emb, transformation_M):
    h = h.astype(jnp.int32)
    r = r.astype(jnp.int32)
    t = t.astype(jnp.int32)
    t_prime = t_prime.astype(jnp.int32)

    m2 = jnp.concatenate([transformation_M, transformation_M], axis=-1)
    re2 = jnp.concatenate(
        [relation_emb, jnp.zeros((NR, RD), jnp.float32)], axis=-1)
    proj3, nrm3 = pl.pallas_call(
        _proj_body,
        grid=(NE // EBLK, NR),
        in_specs=[
            pl.BlockSpec((EBLK, ED), lambda i, j: (i, 0)),
            pl.BlockSpec((1, ED, 2 * RD), lambda i, j: (j, 0, 0)),
            pl.BlockSpec((NR, 2 * RD), lambda i, j: (0, 0)),
        ],
        out_specs=[
            pl.BlockSpec((1, EBLK, 2 * RD), lambda i, j: (j, i, 0)),
            pl.BlockSpec((1, 1, 2048), lambda i, j: (i, 0, 0)),
        ],
        out_shape=[
            jax.ShapeDtypeStruct((NR, NE, 2 * RD), jnp.float32),
            jax.ShapeDtypeStruct((NE // EBLK, 1, 2048), jnp.float32),
        ],
    )(entity_emb, m2, re2)
    proj = proj3.reshape(2 * NR * NE, RD)

    mesh = plsc.VectorSubcoreMesh(core_axis_name="c", subcore_axis_name="s")
    parts = pl.kernel(
        _sc_body,
        mesh=mesh,
        compiler_params=pltpu.CompilerParams(needs_layout_passes=False,
                                             use_tc_tiling_on_sc=False),
        out_type=jax.ShapeDtypeStruct((NW, 64), jnp.float32),
        scratch_types=[
            pltpu.VMEM((TPW,), jnp.int32),
            pltpu.VMEM((TPW,), jnp.int32),
            pltpu.VMEM((TPW,), jnp.int32),
            pltpu.VMEM((TPW,), jnp.int32),
            pltpu.VMEM((CH,), jnp.int32),
            pltpu.VMEM((CH,), jnp.int32),
            pltpu.VMEM((CH,), jnp.int32),
            pltpu.VMEM((CH, RD), jnp.float32),
            pltpu.VMEM((CH, RD), jnp.float32),
            pltpu.VMEM((CH, RD), jnp.float32),
            pltpu.VMEM((CH,), jnp.int32),
            pltpu.VMEM((CH,), jnp.int32),
            pltpu.VMEM((CH,), jnp.int32),
            pltpu.VMEM((CH, RD), jnp.float32),
            pltpu.VMEM((CH, RD), jnp.float32),
            pltpu.VMEM((CH, RD), jnp.float32),
            pltpu.VMEM((5 * 2048,), jnp.float32),
            pltpu.VMEM((64,), jnp.float32),
            pltpu.SemaphoreType.DMA,
            pltpu.SemaphoreType.DMA,
        ],
    )(proj, nrm3.reshape(5 * 2048), h, r, t, t_prime)

    sum_poly = jnp.sum(parts[:, 0:16])
    sum_x = jnp.sum(parts[:, 16:32])
    sum_norm = jnp.sum(parts[:, 32:48])
    loss = (sum_poly - 0.5 * sum_x) / jnp.float32(B * RD)
    reg = sum_norm / jnp.float32(B)
    return (loss + REG_LAMBDA * reg).astype(jnp.float32)
